# baseline (speedup 1.0000x reference)
import functools

import jax
import jax.numpy as jnp
from jax import lax
from jax.experimental import pallas as pl
from jax.experimental.pallas import tpu as pltpu
from jax.experimental.pallas import tpu_sc as plsc

B = 16384
D = 64
LANES = 16
PAD = 17

_info = plsc.get_sparse_core_info()
NC = _info.num_cores
NS = _info.num_subcores
NW = NC * NS
BPW = B // NW
NCHUNK = 4
CHUNK = BPW // NCHUNK

_mesh = plsc.VectorSubcoreMesh(core_axis_name="c", subcore_axis_name="s")


@functools.partial(
    pl.kernel,
    out_type=jax.ShapeDtypeStruct((B,), jnp.float32),
    mesh=_mesh,
    compiler_params=pltpu.CompilerParams(
        needs_layout_passes=False, use_tc_tiling_on_sc=False),
    scratch_types=[
        pltpu.VMEM((NCHUNK, CHUNK), jnp.int32),
        pltpu.VMEM((NCHUNK, CHUNK), jnp.int32),
        pltpu.VMEM((BPW, D), jnp.float32),
        pltpu.VMEM((BPW, D), jnp.float32),
        pltpu.VMEM((BPW * PAD,), jnp.float32),
        pltpu.VMEM((BPW,), jnp.float32),
        pltpu.SemaphoreType.DMA,
        pltpu.SemaphoreType.DMA,
    ],
)
def _sc_dot(user_hbm, item_hbm, uemb_hbm, iemb_hbm, out_hbm,
            uidx, iidx, urows, irows, part, outc, usem, isem):
    wid = lax.axis_index("s") * NC + lax.axis_index("c")
    base = wid * BPW

    pltpu.sync_copy(user_hbm.at[wid], uidx)
    pltpu.sync_copy(item_hbm.at[wid], iidx)

    copies = []
    for c in range(NCHUNK):
        copies.append(pltpu.async_copy(
            uemb_hbm.at[uidx.at[c]], urows.at[pl.ds(c * CHUNK, CHUNK)], usem))
        copies.append(pltpu.async_copy(
            iemb_hbm.at[iidx.at[c]], irows.at[pl.ds(c * CHUNK, CHUNK)], isem))
    for cp in copies:
        cp.wait()

    def row_body(r, carry):
        acc = urows[r, pl.ds(0, LANES)] * irows[r, pl.ds(0, LANES)]
        for c in range(1, D // LANES):
            acc += urows[r, pl.ds(c * LANES, LANES)] * irows[r, pl.ds(c * LANES, LANES)]
        part[pl.ds(r * PAD, LANES)] = acc
        return carry

    lax.fori_loop(0, BPW, row_body, 0, unroll=2)

    def grp_body(g, carry):
        rows = (g * LANES + lax.iota(jnp.int32, LANES)) * PAD
        acc = plsc.load_gather(part, [rows])
        for j in range(1, LANES):
            acc += plsc.load_gather(part, [rows + j])
        outc[pl.ds(g * LANES, LANES)] = acc
        return carry

    lax.fori_loop(0, BPW // LANES, grp_body, 0, unroll=2)

    pltpu.sync_copy(outc, out_hbm.at[pl.ds(base, BPW)])


def kernel(user, item, attr, user_embedding, item_embedding):
    del attr
    user = user.astype(jnp.int32).reshape(NW, NCHUNK, CHUNK)
    item = item.astype(jnp.int32).reshape(NW, NCHUNK, CHUNK)
    return _sc_dot(user, item, user_embedding, item_embedding)
